# baseline (speedup 1.0000x reference)
import functools

import jax
import jax.numpy as jnp
from jax import lax
from jax.experimental import pallas as pl
from jax.experimental.pallas import tpu as pltpu
from jax.experimental.pallas import tpu_sc as plsc

V = 1_000_000
D = 64
SC = 8.0
B = 4096 * 200
NW = 32
ROWS_PER_W = B // NW
CHUNK = 512
K = CHUNK // 128
NCHUNK = ROWS_PER_W // CHUNK

_mesh = plsc.VectorSubcoreMesh(core_axis_name="c", subcore_axis_name="s")


@functools.partial(
    pl.kernel,
    mesh=_mesh,
    compiler_params=pltpu.CompilerParams(
        needs_layout_passes=False, use_tc_tiling_on_sc=False
    ),
    out_type=jax.ShapeDtypeStruct((B * D,), jnp.bfloat16),
    scratch_types=[
        pltpu.VMEM((K, 128), jnp.int32),
        pltpu.VMEM((CHUNK, D), jnp.float32),
        pltpu.VMEM((CHUNK * D,), jnp.bfloat16),
        pltpu.SemaphoreType.DMA,
    ],
)
def _emb(w_hbm, idx_hbm, out_hbm, idx_v, rows_v, out_v, sem):
    wid = lax.axis_index("s") * 2 + lax.axis_index("c")
    iota = lax.iota(jnp.int32, 16)
    ev = iota * 2

    def chunk_body(g, carry):
        row0 = wid * ROWS_PER_W + g * CHUNK
        grp0 = wid * (ROWS_PER_W // 128) + g * K
        pltpu.sync_copy(idx_hbm.at[pl.ds(grp0, K)], idx_v)
        cps = [
            pltpu.async_copy(
                w_hbm.at[idx_v.at[k]], rows_v.at[pl.ds(k * 128, 128)], sem
            )
            for k in range(K)
        ]
        for cp in cps:
            cp.wait()

        def row_body(r, c2):
            re = jnp.full((16,), r, dtype=jnp.int32)
            for h in range(2):
                ce = ev + 32 * h
                evens = plsc.load_gather(rows_v, [re, ce])
                odds = plsc.load_gather(rows_v, [re, ce + 1])
                c = plsc.pack(
                    evens * SC, odds * SC, format=plsc.PackFormat.INTERLEAVED
                )
                out_v[pl.ds(r * D + 32 * h, 32)] = c
            return c2

        lax.fori_loop(0, CHUNK, row_body, 0, unroll=2)
        pltpu.sync_copy(out_v, out_hbm.at[pl.ds(row0 * D, CHUNK * D)])
        return carry

    lax.fori_loop(0, NCHUNK, chunk_body, 0)


def kernel(input, weight):
    idx2 = input.reshape(B // 128, 128)
    out = _emb(weight, idx2)
    return out.reshape(4096, 200, D)


# speedup vs baseline: 1.2676x; 1.2676x over previous
import functools

import jax
import jax.numpy as jnp
from jax import lax
from jax.experimental import pallas as pl
from jax.experimental.pallas import tpu as pltpu
from jax.experimental.pallas import tpu_sc as plsc

V = 1_000_000
D = 64
SC = 8.0
B = 4096 * 200
NW = 32
ROWS_PER_W = B // NW
CHUNK = 256
K = CHUNK // 128
NCHUNK = ROWS_PER_W // CHUNK
NPAIR = NCHUNK // 2

_mesh = plsc.VectorSubcoreMesh(core_axis_name="c", subcore_axis_name="s")


@functools.partial(
    pl.kernel,
    mesh=_mesh,
    compiler_params=pltpu.CompilerParams(
        needs_layout_passes=False, use_tc_tiling_on_sc=False
    ),
    out_type=jax.ShapeDtypeStruct((B, D), jnp.bfloat16),
    scratch_types=[
        pltpu.VMEM((K, 128), jnp.int32),
        pltpu.VMEM((K, 128), jnp.int32),
        pltpu.VMEM((CHUNK, D), jnp.float32),
        pltpu.VMEM((CHUNK, D), jnp.float32),
        pltpu.VMEM((CHUNK, D), jnp.bfloat16),
        pltpu.VMEM((CHUNK, D), jnp.bfloat16),
        pltpu.SemaphoreType.DMA,
        pltpu.SemaphoreType.DMA,
        pltpu.SemaphoreType.DMA,
        pltpu.SemaphoreType.DMA,
    ],
)
def _emb(
    w_hbm, idx_hbm, out_hbm,
    idx_a, idx_b, rows_a, rows_b, out_a, out_b,
    gsem_a, gsem_b, osem_a, osem_b,
):
    wid = lax.axis_index("s") * 2 + lax.axis_index("c")
    iota = lax.iota(jnp.int32, 16)
    ev = iota * 2
    bufs = (
        (idx_a, rows_a, out_a, gsem_a, osem_a),
        (idx_b, rows_b, out_b, gsem_b, osem_b),
    )

    def fire(g, slot):
        idx_v, rows_v, _, gsem, _ = bufs[slot]
        grp0 = wid * (ROWS_PER_W // 128) + g * K
        pltpu.sync_copy(idx_hbm.at[pl.ds(grp0, K)], idx_v)
        for k in range(K):
            pltpu.async_copy(
                w_hbm.at[idx_v.at[k]], rows_v.at[pl.ds(k * 128, 128)], gsem
            )

    def drain_gather(slot):
        _, rows_v, _, gsem, _ = bufs[slot]
        for k in range(K):
            pltpu.make_async_copy(
                w_hbm.at[pl.ds(0, 128)], rows_v.at[pl.ds(k * 128, 128)], gsem
            ).wait()

    def drain_store(slot):
        _, _, out_v, _, osem = bufs[slot]
        pltpu.make_async_copy(out_hbm.at[pl.ds(0, CHUNK)], out_v, osem).wait()

    def compute_store(g, slot):
        _, rows_v, out_v, _, osem = bufs[slot]

        @plsc.parallel_loop(0, CHUNK, 1, unroll=8)
        def _row(r):
            re = jnp.full((16,), r, dtype=jnp.int32)
            for h in range(2):
                ce = ev + 32 * h
                a = plsc.load_gather(rows_v, [re, ce])
                b = plsc.load_gather(rows_v, [re, ce + 1])
                out_v[r, pl.ds(32 * h, 32)] = plsc.pack(
                    a * SC, b * SC, format=plsc.PackFormat.INTERLEAVED
                )

        row0 = wid * ROWS_PER_W + g * CHUNK
        pltpu.async_copy(out_v, out_hbm.at[pl.ds(row0, CHUNK)], osem)

    fire(0, 0)
    fire(1, 1)

    def pair_body(p, carry):
        for slot in range(2):
            g = 2 * p + slot
            drain_gather(slot)
            pl.when(p > 0)(lambda slot=slot: drain_store(slot))
            compute_store(g, slot)
            pl.when(p < NPAIR - 1)(lambda g=g, slot=slot: fire(g + 2, slot))
        return carry

    lax.fori_loop(0, NPAIR, pair_body, 0)
    drain_store(0)
    drain_store(1)


def kernel(input, weight):
    idx2 = input.reshape(B // 128, 128)
    out = _emb(weight, idx2)
    return out.reshape(4096, 200, D)
